# baseline (speedup 1.0000x reference)
import functools

import jax
import jax.numpy as jnp
from jax import lax
from jax.experimental import pallas as pl
from jax.experimental.pallas import tpu as pltpu
from jax.experimental.pallas import tpu_sc as plsc

N_PATCHES = 200
AH = AW = 128
H = W = 2048
C = 3
PATCH = 32
SY = H // AH
CHUNK = 16
WPC = W // CHUNK
PATLEN = C * PATCH * 2
ROWS = 2 * N_PATCHES * PATLEN
NC, NS = 2, 16
NW = NC * NS
RPW = ROWS // NW
GCH = 120
NGC = RPW // GCH
KPAD = 256


def _topk_body(score_ref, p_ref, sa_ref, ridx_ref, s_ref):
    s_ref[...] = score_ref[...]
    P = jnp.max(score_ref[...].reshape(2, AH // 8, 8, AW), axis=2)
    iv = lax.broadcasted_iota(jnp.int32, (AH // 8, AW), 0)
    r8 = lax.broadcasted_iota(jnp.int32, (8, AW), 0)
    c8 = lax.broadcasted_iota(jnp.int32, (8, AW), 1)
    pos8 = r8 * AW + c8
    lane = lax.broadcasted_iota(jnp.int32, (KPAD,), 0)

    def step(j, b, P, idxv, sav):
        Pb = P[b]
        m = jnp.max(Pb)
        v = jnp.min(jnp.where(Pb == m, iv, jnp.int32(AH // 8)))
        row0 = pl.multiple_of(v * 8, 8)
        blk = s_ref[b, pl.ds(row0, 8), :]
        chosen = v * (8 * AW) + jnp.min(
            jnp.where(blk == m, pos8, jnp.int32(1 << 30)))
        hit = pos8 == (chosen - v * (8 * AW))
        pv = jnp.sum(jnp.where(hit, p_ref[b, pl.ds(row0, 8), :], 0.0))
        blk = jnp.where(hit, jnp.float32(-1e30), blk)
        s_ref[b, pl.ds(row0, 8), :] = blk
        Pb = jnp.where(iv == v, jnp.max(blk, axis=0)[None, :], Pb)
        P = jnp.where(lax.broadcasted_iota(jnp.int32, (2, 1, 1), 0) == b,
                      Pb[None], P)
        idxv = jnp.where(lane == j, chosen, idxv)
        sav = jnp.where(lane == j, pv, sav)
        return P, idxv, sav

    def body(j, st):
        P, i0, v0, i1, v1 = st
        P, i0, v0 = step(j, 0, P, i0, v0)
        P, i1, v1 = step(j, 1, P, i1, v1)
        return P, i0, v0, i1, v1

    z_i = jnp.zeros((KPAD,), jnp.int32)
    z_f = jnp.zeros((KPAD,), jnp.float32)
    _, i0, v0, i1, v1 = lax.fori_loop(
        0, N_PATCHES, body, (P, z_i, z_f, z_i, z_f))

    w = lax.broadcasted_iota(jnp.int32, (PATLEN,), 0)
    c = w // (PATCH * 2)
    rem = w - c * (PATCH * 2)
    r = rem // 2
    k = rem - r * 2
    pat = c * (H * WPC) + r * WPC + k
    for b, (idxv, sav) in enumerate(((i0, v0), (i1, v1))):
        ys = idxv // AW
        xs = idxv - ys * AW
        y0 = jnp.minimum(ys * SY, H - PATCH)
        x0c = jnp.minimum(xs, WPC - 2)
        off = b * (C * H * WPC) + y0 * WPC + x0c
        ridx_ref[b] = off[:, None] + pat[None, :]
        sa_ref[b, 0] = sav


def _topk_call(score, p):
    return pl.pallas_call(
        _topk_body,
        out_shape=[jax.ShapeDtypeStruct((2, 1, KPAD), jnp.float32),
                   jax.ShapeDtypeStruct((2, KPAD, PATLEN), jnp.int32)],
        scratch_shapes=[pltpu.VMEM((2, AH, AW), jnp.float32)],
    )(score, p)


@functools.cache
def _make_gather():
    mesh = plsc.VectorSubcoreMesh(core_axis_name="c", subcore_axis_name="s")

    @functools.partial(
        pl.kernel,
        mesh=mesh,
        out_type=jax.ShapeDtypeStruct((ROWS, CHUNK), jnp.float32),
        compiler_params=pltpu.CompilerParams(use_tc_tiling_on_sc=False),
        scratch_types=[
            pltpu.VMEM((RPW,), jnp.int32),
            pltpu.VMEM((RPW, CHUNK), jnp.float32),
            pltpu.SemaphoreType.DMA,
        ],
    )
    def gather_k(table_hbm, ridx_hbm, out_hbm, idx_v, rows_v, sem):
        wid = lax.axis_index("s") * NC + lax.axis_index("c")
        base = wid * RPW
        pltpu.sync_copy(ridx_hbm.at[pl.ds(base, RPW)], idx_v)
        cps = [
            pltpu.async_copy(table_hbm.at[idx_v.at[pl.ds(g * GCH, GCH)]],
                             rows_v.at[pl.ds(g * GCH, GCH)], sem)
            for g in range(NGC)
        ]
        for cp in cps:
            cp.wait()
        pltpu.sync_copy(rows_v, out_hbm.at[pl.ds(base, RPW)])

    return gather_k


def kernel(x_low, x_high, attention, WSI):
    B = attention.shape[0]
    flat = attention.reshape(B, -1)
    p = flat / jnp.sum(flat, axis=-1, keepdims=True)
    logp = jnp.log(p + 1e-12)
    u = jax.random.uniform(jax.random.key(42), flat.shape,
                           minval=1e-9, maxval=1.0)
    gumbel = -jnp.log(-jnp.log(u))
    score = logp + gumbel
    sa_pad, ridx_pad = _topk_call(score.reshape(B, AH, AW),
                                  p.reshape(B, AH, AW))
    ridx = ridx_pad[:, :N_PATCHES, :].reshape(-1)
    table = WSI.reshape(B * C * H * WPC, CHUNK)
    rows = _make_gather()(table, ridx)
    patches = rows.reshape(B, N_PATCHES, C, PATCH, PATCH)
    return patches, sa_pad[:, 0, :N_PATCHES]


# speedup vs baseline: 1.0846x; 1.0846x over previous
import functools

import jax
import jax.numpy as jnp
from jax import lax
from jax.experimental import pallas as pl
from jax.experimental.pallas import tpu as pltpu
from jax.experimental.pallas import tpu_sc as plsc

N_PATCHES = 200
AH = AW = 128
H = W = 2048
C = 3
PATCH = 32
SY = H // AH
NC, NS = 2, 16
NW = NC * NS
UNITS = 2 * N_PATCHES * C
UPW = 38
UPAD = NW * UPW
DROWS = 40
BLKW = 256
KPAD = 256


def _topk_body(score_ref, p_ref, sa_ref, idx_ref):
    pos = (lax.broadcasted_iota(jnp.int32, (AH, AW), 0) * AW
           + lax.broadcasted_iota(jnp.int32, (AH, AW), 1))
    lane = lax.broadcasted_iota(jnp.int32, (KPAD,), 0)

    def step(j, s, pb, idxv, sav):
        m = jnp.max(s)
        chosen = jnp.min(jnp.where(s == m, pos, jnp.int32(1 << 30)))
        hit = pos == chosen
        pv = jnp.sum(jnp.where(hit, pb, jnp.float32(0.0)))
        s = jnp.where(hit, jnp.float32(-1e30), s)
        idxv = jnp.where(lane == j, chosen, idxv)
        sav = jnp.where(lane == j, pv, sav)
        return s, idxv, sav

    def body(j, st):
        s0, s1, i0, i1, a0, a1 = st
        s0, i0, a0 = step(j, s0, p_ref[0], i0, a0)
        s1, i1, a1 = step(j, s1, p_ref[1], i1, a1)
        return s0, s1, i0, i1, a0, a1

    z_i = jnp.zeros((KPAD,), jnp.int32)
    z_f = jnp.zeros((KPAD,), jnp.float32)
    _, _, i0, i1, a0, a1 = lax.fori_loop(
        0, N_PATCHES, body,
        (score_ref[0], score_ref[1], z_i, z_i, z_f, z_f))

    idx_ref[0, 0] = i0
    idx_ref[1, 0] = i1
    sa_ref[0, 0] = a0
    sa_ref[1, 0] = a1


def _topk_call(score, p):
    return pl.pallas_call(
        _topk_body,
        out_shape=[jax.ShapeDtypeStruct((2, 1, KPAD), jnp.float32),
                   jax.ShapeDtypeStruct((2, 1, KPAD), jnp.int32)],
    )(score, p)


@functools.cache
def _make_gather():
    mesh = plsc.VectorSubcoreMesh(core_axis_name="c", subcore_axis_name="s")

    @functools.partial(
        pl.kernel,
        mesh=mesh,
        out_type=jax.ShapeDtypeStruct((UPAD, PATCH, PATCH), jnp.float32),
        compiler_params=pltpu.CompilerParams(use_tc_tiling_on_sc=True),
        scratch_types=[
            pltpu.VMEM((DROWS, 128), jnp.int32),
            pltpu.VMEM((PATCH, BLKW), jnp.float32),
            pltpu.VMEM((PATCH, BLKW), jnp.float32),
            pltpu.VMEM((PATCH, PATCH), jnp.float32),
            pltpu.VMEM((PATCH, PATCH), jnp.float32),
            pltpu.SemaphoreType.DMA,
            pltpu.SemaphoreType.DMA,
            pltpu.SemaphoreType.DMA,
            pltpu.SemaphoreType.DMA,
        ],
    )
    def gather_k(wsi_hbm, desc_hbm, out_hbm, desc_v, buf0, buf1,
                 pbuf0, pbuf1, sr0, sr1, sw0, sw1):
        wid = lax.axis_index("s") * NC + lax.axis_index("c")
        pltpu.sync_copy(desc_hbm.at[wid], desc_v)
        lane16 = lax.broadcasted_iota(jnp.int32, (16,), 0)
        bufs = (buf0, buf1)
        pbufs = (pbuf0, pbuf1)
        srs = (sr0, sr1)
        sws = (sw0, sw1)

        def fields(t):
            v = desc_v[t, pl.ds(0, 16)]
            return v[0], v[1], v[2], v[3]

        def start_read(t, buf, sem):
            row0, xa, _, _ = fields(t)
            row0 = pl.multiple_of(row0, 16)
            xa = pl.multiple_of(xa, 128)
            return pltpu.async_copy(
                wsi_hbm.at[pl.ds(row0, PATCH), pl.ds(xa, BLKW)], buf, sem)

        reads = [start_read(0, buf0, sr0), start_read(1, buf1, sr1)]
        writes = [None, None]
        for t in range(UPW):
            pipe = t % 2
            buf = bufs[pipe]
            pbuf = pbufs[pipe]
            reads[pipe].wait()
            if writes[pipe] is not None:
                writes[pipe].wait()
            _, _, xoff, uout = fields(t)
            xoff = pl.multiple_of(xoff, 16)
            for r in range(PATCH):
                for h in range(2):
                    pbuf[r, pl.ds(h * 16, 16)] = (
                        buf[r, pl.ds(xoff + h * 16, 16)])
            writes[pipe] = pltpu.async_copy(
                pbuf, out_hbm.at[uout], sws[pipe])
            if t + 2 < UPW:
                reads[pipe] = start_read(t + 2, buf, srs[pipe])
        writes[0].wait()
        writes[1].wait()

    return gather_k


def kernel(x_low, x_high, attention, WSI):
    B = attention.shape[0]
    flat = attention.reshape(B, -1)
    p = flat / jnp.sum(flat, axis=-1, keepdims=True)
    logp = jnp.log(p + 1e-12)
    u = jax.random.uniform(jax.random.key(42), flat.shape,
                           minval=1e-9, maxval=1.0)
    gumbel = -jnp.log(-jnp.log(u))
    score = logp + gumbel
    sa_pad, idx_pad = _topk_call(score.reshape(B, AH, AW),
                                 p.reshape(B, AH, AW))

    idx_flat = idx_pad.reshape(B, KPAD)
    uu = jnp.arange(UPAD, dtype=jnp.int32)
    bb = jnp.minimum(uu // (N_PATCHES * C), B - 1)
    nn = (uu % (N_PATCHES * C)) // C
    cc = uu % C
    cell = idx_flat[bb, nn]
    ys = cell // AW
    xs = cell % AW
    y0 = jnp.minimum(ys * SY, H - PATCH)
    x0 = jnp.minimum(xs * SY, W - PATCH)
    xa = jnp.minimum((x0 // 128) * 128, W - BLKW)
    xoff = x0 - xa
    row0 = (bb * C + cc) * H + y0
    fields = jnp.stack([row0, xa, xoff, uu], axis=-1).astype(jnp.int32)
    desc = jnp.zeros((NW, DROWS, 128), jnp.int32)
    desc = desc.at[:, :UPW, :4].set(fields.reshape(NW, UPW, 4))

    out3 = _make_gather()(WSI.reshape(B * C * H, W), desc)
    patches = out3[:UNITS].reshape(B, N_PATCHES, C, PATCH, PATCH)
    return patches, sa_pad[:, 0, :N_PATCHES]
